# baseline (speedup 1.0000x reference)
import functools

import jax
import jax.numpy as jnp
from jax import lax
from jax.experimental import pallas as pl
from jax.experimental.pallas import tpu as pltpu
from jax.experimental.pallas import tpu_sc as plsc

_VOCAB = 1_000_000
_DIM = 64
_N_FIXED = 500_000

_NC = 2
_NS = 16
_NW = _NC * _NS

_SEG = 128
_SEGS_PER_ITER = 8


def _gather_kernel(n_idx: int):
    k_per_w = n_idx // _NW
    c_per_iter = _SEG * _SEGS_PER_ITER
    n_iter = k_per_w // c_per_iter
    assert n_iter * c_per_iter == k_per_w

    mesh = plsc.VectorSubcoreMesh(core_axis_name="c", subcore_axis_name="s")

    @functools.partial(
        pl.kernel,
        out_type=jax.ShapeDtypeStruct((n_idx, _DIM), jnp.float32),
        mesh=mesh,
        scratch_types=[
            pltpu.VMEM((_SEGS_PER_ITER, _SEG), jnp.int32),
            pltpu.VMEM((c_per_iter, _DIM), jnp.float32),
            pltpu.SemaphoreType.DMA,
            pltpu.SemaphoreType.DMA,
        ],
        compiler_params=pltpu.CompilerParams(use_tc_tiling_on_sc=False),
    )
    def k(emb_hbm, idx_hbm, out_hbm, idx_v, rows_v, sem_i, sem_g):
        wid = lax.axis_index("s") * _NC + lax.axis_index("c")
        base_row = wid * (k_per_w // _SEG)

        def body(it, carry):
            row0 = base_row + it * _SEGS_PER_ITER
            pltpu.async_copy(
                idx_hbm.at[pl.ds(row0, _SEGS_PER_ITER)], idx_v, sem_i
            ).wait()
            handles = []
            for g in range(_SEGS_PER_ITER):
                handles.append(
                    pltpu.async_copy(
                        emb_hbm.at[idx_v.at[g]],
                        rows_v.at[pl.ds(g * _SEG, _SEG)],
                        sem_g,
                    )
                )
            for h in handles:
                h.wait()
            pltpu.sync_copy(
                rows_v, out_hbm.at[pl.ds(row0 * _SEG, c_per_iter)]
            )
            return carry

        lax.fori_loop(0, n_iter, body, 0)

    return k


def kernel(inputs, fixed_w, var_w):
    b, h = inputs.shape
    n_idx = b * h
    idx2d = inputs.reshape(n_idx // _SEG, _SEG).astype(jnp.int32)
    emb = jnp.concatenate([fixed_w, var_w], axis=0)
    out = _gather_kernel(n_idx)(emb, idx2d)
    return out.reshape(b, h, _DIM)


# speedup vs baseline: 1.1876x; 1.1876x over previous
import functools

import jax
import jax.numpy as jnp
from jax import lax
from jax.experimental import pallas as pl
from jax.experimental.pallas import tpu as pltpu
from jax.experimental.pallas import tpu_sc as plsc

_DIM = 64
_NC = 2
_NS = 16
_NW = _NC * _NS

_SEG = 128
_SEG_SHIFT = 7
_L = 16


def _routed_gather(n_idx: int, n_fixed: int, dim: int):
    k_per_w = n_idx // _NW
    assert k_per_w * _NW == n_idx and k_per_w % _SEG == 0
    n_groups = k_per_w // _L
    cap = k_per_w + 4 * _SEG

    mesh = plsc.VectorSubcoreMesh(core_axis_name="c", subcore_axis_name="s")

    @functools.partial(
        pl.kernel,
        out_type=jax.ShapeDtypeStruct((n_idx, dim), jnp.float32),
        mesh=mesh,
        scratch_types=[
            pltpu.VMEM((k_per_w,), jnp.int32),
            pltpu.VMEM((cap,), jnp.int32),
            pltpu.VMEM((cap,), jnp.int32),
            pltpu.VMEM((2, _SEG, dim), jnp.float32),
            pltpu.SemaphoreType.DMA,
            pltpu.SemaphoreType.DMA,
            pltpu.SemaphoreType.DMA,
        ],
        compiler_params=pltpu.CompilerParams(
            use_tc_tiling_on_sc=False, needs_layout_passes=False),
    )
    def k(fixed_hbm, var_hbm, idx_hbm, out_hbm,
          idx_v, cidx, cpos, rows, sem_i, sem_g, sem_s):
        wid = lax.axis_index("s") * _NC + lax.axis_index("c")
        base = wid * k_per_w
        pltpu.async_copy(idx_hbm.at[pl.ds(base, k_per_w)], idx_v, sem_i).wait()

        iota = lax.iota(jnp.int32, _L)
        nfix = jnp.int32(n_fixed)
        one = jnp.full((_L,), 1, jnp.int32)
        zero = jnp.full((_L,), 0, jnp.int32)

        def count_body(g, acc):
            v = idx_v[pl.ds(g * _L, _L)]
            return acc + jnp.where(v < nfix, one, zero)

        cnt = lax.fori_loop(0, n_groups, count_body,
                            jnp.zeros((_L,), jnp.int32))
        nf = jnp.sum(cnt)
        vstart = ((nf + (_SEG - 1)) & jnp.int32(-_SEG)) + _SEG

        def compact_body(g, carry):
            foff, voff = carry
            v = idx_v[pl.ds(g * _L, _L)]
            m = v < nfix
            pos = (base + g * _L) + iota
            plsc.store_compressed(cidx.at[pl.ds(foff, _L)], v, mask=m)
            plsc.store_compressed(cpos.at[pl.ds(foff, _L)], pos, mask=m)
            nm = jnp.logical_not(m)
            plsc.store_compressed(cidx.at[pl.ds(voff, _L)], v - nfix, mask=nm)
            plsc.store_compressed(cpos.at[pl.ds(voff, _L)], pos, mask=nm)
            c = jnp.sum(jnp.where(m, one, zero))
            return foff + c, voff + (_L - c)

        nf2, vend = lax.fori_loop(0, n_groups, compact_body,
                                  (jnp.int32(0), vstart))
        nv = vend - vstart

        def pad_tail(end):
            @pl.when(end > 0)
            def _():
                last = jnp.full((_L,), end - 1, jnp.int32)
                li = plsc.load_gather(cidx, [last])
                lp = plsc.load_gather(cpos, [last])
                for j in range(_SEG // _L):
                    cidx[pl.ds(end + j * _L, _L)] = li
                    cpos[pl.ds(end + j * _L, _L)] = lp

        pad_tail(nf2)
        pad_tail(vend)

        n_fseg = ((nf2 + (_SEG - 1)) & jnp.int32(-_SEG)) >> _SEG_SHIFT
        n_vseg = ((nv + (_SEG - 1)) & jnp.int32(-_SEG)) >> _SEG_SHIFT

        def run_table(table_hbm, seg0, n_seg):
            def gather(s):
                off = (seg0 + s) * _SEG
                return pltpu.make_async_copy(
                    table_hbm.at[cidx.at[pl.ds(off, _SEG)]],
                    rows.at[s % 2], sem_g)

            @pl.when(n_seg > 0)
            def _():
                gather(jnp.int32(0)).start()

            def body(s, carry):
                gather(s).wait()

                @pl.when(s + 1 < n_seg)
                def _():
                    gather(s + 1).start()

                off = (seg0 + s) * _SEG
                pltpu.async_copy(
                    rows.at[s % 2],
                    out_hbm.at[cpos.at[pl.ds(off, _SEG)]], sem_s).wait()
                return carry

            lax.fori_loop(0, n_seg, body, 0)

        run_table(fixed_hbm, jnp.int32(0), n_fseg)
        run_table(var_hbm, vstart >> _SEG_SHIFT, n_vseg)

    return k


def kernel(inputs, fixed_w, var_w):
    b, h = inputs.shape
    n_idx = b * h
    idx1d = inputs.reshape(n_idx).astype(jnp.int32)
    out = _routed_gather(n_idx, fixed_w.shape[0], _DIM)(fixed_w, var_w, idx1d)
    return out.reshape(b, h, _DIM)


# speedup vs baseline: 1.5453x; 1.3012x over previous
import functools

import jax
import jax.numpy as jnp
from jax import lax
from jax.experimental import pallas as pl
from jax.experimental.pallas import tpu as pltpu
from jax.experimental.pallas import tpu_sc as plsc

_DIM = 64
_NC = 2
_NS = 16
_NW = _NC * _NS

_SEG = 128
_SEG_SHIFT = 7
_L = 16


def _routed_gather(n_idx: int, n_fixed: int, dim: int):
    k_per_w = n_idx // _NW
    assert k_per_w * _NW == n_idx and k_per_w % _SEG == 0
    n_groups = k_per_w // _L
    cap = k_per_w + 4 * _SEG

    mesh = plsc.VectorSubcoreMesh(core_axis_name="c", subcore_axis_name="s")

    @functools.partial(
        pl.kernel,
        out_type=jax.ShapeDtypeStruct((2 * n_idx, dim), jnp.float32),
        mesh=mesh,
        scratch_types=[
            pltpu.VMEM((k_per_w,), jnp.int32),
            pltpu.VMEM((cap,), jnp.int32),
            pltpu.VMEM((cap,), jnp.int32),
            pltpu.VMEM((2, _SEG, dim), jnp.float32),
            pltpu.SemaphoreType.DMA,
            pltpu.SemaphoreType.DMA,
            pltpu.SemaphoreType.DMA,
        ],
        compiler_params=pltpu.CompilerParams(
            use_tc_tiling_on_sc=False, needs_layout_passes=False),
    )
    def k(fixed_t, var_t, idx_hbm, out_t,
          idx_v, cidx, cpos, rows, sem_i, sem_g, sem_s):
        wid = lax.axis_index("s") * _NC + lax.axis_index("c")
        base = wid * k_per_w
        pltpu.async_copy(idx_hbm.at[pl.ds(base, k_per_w)], idx_v, sem_i).wait()

        iota = lax.iota(jnp.int32, _L)
        nfix = jnp.int32(n_fixed)
        one = jnp.full((_L,), 1, jnp.int32)
        zero = jnp.full((_L,), 0, jnp.int32)

        def count_body(g, acc):
            v = idx_v[pl.ds(g * _L, _L)]
            return acc + jnp.where(v < nfix, one, zero)

        cnt = lax.fori_loop(0, n_groups, count_body,
                            jnp.zeros((_L,), jnp.int32))
        nf = jnp.sum(cnt)
        vstart = ((nf + (_SEG - 1)) & jnp.int32(-_SEG)) + _SEG

        def compact_body(g, carry):
            foff, voff = carry
            v = idx_v[pl.ds(g * _L, _L)]
            m = v < nfix
            pos = 2 * ((base + g * _L) + iota)
            plsc.store_compressed(cidx.at[pl.ds(foff, _L)], v, mask=m)
            plsc.store_compressed(cpos.at[pl.ds(foff, _L)], pos, mask=m)
            nm = jnp.logical_not(m)
            plsc.store_compressed(cidx.at[pl.ds(voff, _L)], v - nfix, mask=nm)
            plsc.store_compressed(cpos.at[pl.ds(voff, _L)], pos, mask=nm)
            c = jnp.sum(jnp.where(m, one, zero))
            return foff + c, voff + (_L - c)

        nf2, vend = lax.fori_loop(0, n_groups, compact_body,
                                  (jnp.int32(0), vstart))
        nv = vend - vstart

        def pad_tail(end):
            @pl.when(end > 0)
            def _():
                last = jnp.full((_L,), end - 1, jnp.int32)
                li = plsc.load_gather(cidx, [last])
                lp = plsc.load_gather(cpos, [last])
                for j in range(_SEG // _L):
                    cidx[pl.ds(end + j * _L, _L)] = li
                    cpos[pl.ds(end + j * _L, _L)] = lp

        pad_tail(nf2)
        pad_tail(vend)

        n_fseg = ((nf2 + (_SEG - 1)) & jnp.int32(-_SEG)) >> _SEG_SHIFT
        n_vseg = ((nv + (_SEG - 1)) & jnp.int32(-_SEG)) >> _SEG_SHIFT

        def run_table(table, seg0, n_seg):
            def gather(s):
                off = (seg0 + s) * _SEG
                return pltpu.make_async_copy(
                    table.at[cidx.at[pl.ds(off, _SEG)]],
                    rows.at[s % 2], sem_g)

            @pl.when(n_seg > 0)
            def _():
                gather(jnp.int32(0)).start()

            def body(s, carry):
                gather(s).wait()

                @pl.when(s + 1 < n_seg)
                def _():
                    gather(s + 1).start()

                off = (seg0 + s) * _SEG
                pltpu.async_copy(
                    rows.at[s % 2],
                    out_t.at[cpos.at[pl.ds(off, _SEG)]], sem_s).wait()
                return carry

            lax.fori_loop(0, n_seg, body, 0)

        run_table(fixed_t, jnp.int32(0), n_fseg)
        run_table(var_t, vstart >> _SEG_SHIFT, n_vseg)

    return k


def kernel(inputs, fixed_w, var_w):
    b, h = inputs.shape
    n_idx = b * h
    n_fixed = fixed_w.shape[0]
    idx1d = inputs.reshape(n_idx).astype(jnp.int32)
    fixed2 = jax.lax.optimization_barrier(
        fixed_w.reshape(n_fixed // 2, 2 * _DIM)).reshape(n_fixed, _DIM)
    var2 = jax.lax.optimization_barrier(
        var_w.reshape(var_w.shape[0] // 2, 2 * _DIM)).reshape(
            var_w.shape[0], _DIM)
    out3 = _routed_gather(n_idx, n_fixed, _DIM)(fixed2, var2, idx1d)
    return out3.reshape(n_idx, 2 * _DIM)[:, :_DIM].reshape(b, h, _DIM)
